# Initial kernel scaffold; baseline (speedup 1.0000x reference)
import jax
import jax.numpy as jnp
from jax.experimental import pallas as pl


def kernel(sents, table):
    raise NotImplementedError("write your pallas kernel here")


# speedup vs baseline: 1.4253x; 1.4253x over previous
import functools

import jax
import jax.numpy as jnp
from jax import lax
from jax.experimental import pallas as pl
from jax.experimental.pallas import tpu as pltpu
from jax.experimental.pallas import tpu_sc as plsc

EMBED_DIM = 32
CHUNK = 128
NUM_WORKERS = 32


@functools.partial(jax.jit, static_argnames=())
def _sc_embedding_lookup(idx2d, table):
    n_rows_total = idx2d.shape[0] * idx2d.shape[1]
    chunks_per_w = idx2d.shape[0] // NUM_WORKERS

    mesh = plsc.VectorSubcoreMesh(core_axis_name="c", subcore_axis_name="s")

    @functools.partial(
        pl.kernel,
        mesh=mesh,
        out_type=jax.ShapeDtypeStruct((n_rows_total, EMBED_DIM), jnp.float32),
        scratch_types=[
            pltpu.VMEM((chunks_per_w, CHUNK), jnp.int32),
            pltpu.VMEM((2, CHUNK, EMBED_DIM), jnp.float32),
            pltpu.SemaphoreType.DMA,
        ],
        compiler_params=pltpu.CompilerParams(use_tc_tiling_on_sc=False),
    )
    def k(table_hbm, idx_hbm, out_hbm, idx_v, rows_v, gsem):
        wid = lax.axis_index("s") * 2 + lax.axis_index("c")
        chunk0 = wid * chunks_per_w

        pltpu.sync_copy(idx_hbm.at[pl.ds(chunk0, chunks_per_w)], idx_v)

        def start_gather(j, b):
            pltpu.async_copy(table_hbm.at[idx_v.at[j]], rows_v.at[b], gsem)

        def wait_gather(b):
            pltpu.make_async_copy(
                table_hbm.at[idx_v.at[0]], rows_v.at[b], gsem
            ).wait()

        start_gather(0, 0)

        def outer(g, carry):
            for b in range(2):
                j = 2 * g + b

                @pl.when(j + 1 < chunks_per_w)
                def _():
                    start_gather(j + 1, 1 - b)

                wait_gather(b)
                pltpu.sync_copy(
                    rows_v.at[b],
                    out_hbm.at[pl.ds((chunk0 + j) * CHUNK, CHUNK)],
                )
            return carry

        lax.fori_loop(0, chunks_per_w // 2, outer, 0)

    return k(table, idx2d)


def kernel(sents, table):
    if sents.ndim < 2:
        sents = sents[None, :]
    batch, seq_len = sents.shape
    n = batch * seq_len
    idx2d = sents.reshape(n // CHUNK, CHUNK).astype(jnp.int32)
    out = _sc_embedding_lookup(idx2d, table)
    return out.reshape(batch, seq_len, EMBED_DIM)


# speedup vs baseline: 1.5026x; 1.0542x over previous
import functools

import jax
import jax.numpy as jnp
from jax import lax
from jax.experimental import pallas as pl
from jax.experimental.pallas import tpu as pltpu
from jax.experimental.pallas import tpu_sc as plsc

EMBED_DIM = 32
CHUNK = 128
NUM_WORKERS = 32
NBUF = 8


@functools.partial(jax.jit, static_argnames=())
def _sc_embedding_lookup(idx2d, table):
    n_rows_total = idx2d.shape[0] * idx2d.shape[1]
    chunks_per_w = idx2d.shape[0] // NUM_WORKERS

    mesh = plsc.VectorSubcoreMesh(core_axis_name="c", subcore_axis_name="s")

    @functools.partial(
        pl.kernel,
        mesh=mesh,
        out_type=jax.ShapeDtypeStruct((n_rows_total, EMBED_DIM), jnp.float32),
        scratch_types=[
            pltpu.VMEM((chunks_per_w, CHUNK), jnp.int32),
            pltpu.VMEM((NBUF, CHUNK, EMBED_DIM), jnp.float32),
            pltpu.SemaphoreType.DMA,
        ],
        compiler_params=pltpu.CompilerParams(use_tc_tiling_on_sc=False),
    )
    def k(table_hbm, idx_hbm, out_hbm, idx_v, rows_v, gsem):
        wid = lax.axis_index("s") * 2 + lax.axis_index("c")
        chunk0 = wid * chunks_per_w

        pltpu.sync_copy(idx_hbm.at[pl.ds(chunk0, chunks_per_w)], idx_v)

        def start_gather(j, b):
            pltpu.async_copy(table_hbm.at[idx_v.at[j]], rows_v.at[b], gsem)

        def wait_gather(b):
            pltpu.make_async_copy(
                table_hbm.at[idx_v.at[0]], rows_v.at[b], gsem
            ).wait()

        for b in range(NBUF):
            start_gather(b, b)

        def outer(g, carry):
            for b in range(NBUF):
                j = g * NBUF + b
                wait_gather(b)
                pltpu.sync_copy(
                    rows_v.at[b],
                    out_hbm.at[pl.ds((chunk0 + j) * CHUNK, CHUNK)],
                )

                @pl.when(j + NBUF < chunks_per_w)
                def _():
                    start_gather(j + NBUF, b)

            return carry

        lax.fori_loop(0, chunks_per_w // NBUF, outer, 0)

    return k(table, idx2d)


def kernel(sents, table):
    if sents.ndim < 2:
        sents = sents[None, :]
    batch, seq_len = sents.shape
    n = batch * seq_len
    idx2d = sents.reshape(n // CHUNK, CHUNK).astype(jnp.int32)
    out = _sc_embedding_lookup(idx2d, table)
    return out.reshape(batch, seq_len, EMBED_DIM)
